# Initial kernel scaffold; baseline (speedup 1.0000x reference)
import jax
import jax.numpy as jnp
from jax.experimental import pallas as pl


def kernel(pos, edge_index0, precomp0, connection0, edge_index1, precomp1, connection1, edge_index2, precomp2, connection2, edge_index3, precomp3, connection3, pool_idx1, pool_conn1, pool_idx2, pool_conn2, pool_idx3, pool_conn3, unpool3, unpool2, unpool1, b01_W1, b01_b1, b01_nb1, b01_W2, b01_b2, b01_nb2, b01_Wres, b11_W1, b11_b1, b11_nb1, b11_W2, b11_b2, b11_nb2, b11_Wres, b12_W1, b12_b1, b12_nb1, b12_W2, b12_b2, b12_nb2, b21_W1, b21_b1, b21_nb1, b21_W2, b21_b2, b21_nb2, b21_Wres, b22_W1, b22_b1, b22_nb1, b22_W2, b22_b2, b22_nb2, b31_W1, b31_b1, b31_nb1, b31_W2, b31_b2, b31_nb2, b32_W1, b32_b1, b32_nb1, b32_W2, b32_b2, b32_nb2, b41_W1, b41_b1, b41_nb1, b41_W2, b41_b2, b41_nb2, b42_W1, b42_b1, b42_nb1, b42_W2, b42_b2, b42_nb2, b51_W1, b51_b1, b51_nb1, b51_W2, b51_b2, b51_nb2, b52_W1, b52_b1, b52_nb1, b52_W2, b52_b2, b52_nb2, b61_W1, b61_b1, b61_nb1, b61_W2, b61_b2, b61_nb2, b61_Wres, b62_W1, b62_b1, b62_nb1, b62_W2, b62_b2, b62_nb2, b71_W1, b71_b1, b71_nb1, b71_W2, b71_b2, b71_nb2, b71_Wres, b72_W1, b72_b1, b72_nb1, b72_W2, b72_b2, b72_nb2, b81_W1, b81_b1, b81_nb1, b81_W2, b81_b2, b81_nb2, b81_Wres, b82_W1, b82_b1, b82_nb1, b82_W2, b82_b2, b82_nb2, lin1_W, nonlin1_b):
    raise NotImplementedError("write your pallas kernel here")


# speedup vs baseline: 1.0001x; 1.0001x over previous
import functools

import jax
import jax.numpy as jnp
from jax.experimental import pallas as pl


def _cmul(a, b):
    ar, ai = a[..., 0], a[..., 1]
    br, bi = b[..., 0], b[..., 1]
    return jnp.stack([ar * br - ai * bi, ar * bi + ai * br], axis=-1)


def _c_nonlin(x, bias):
    mag = jnp.sqrt(jnp.sum(x * x, axis=-1) + 1e-12)
    scale = jax.nn.relu(mag + bias) / (mag + 1e-6)
    return x * scale[..., None]


def _harmonic_conv(x, ei, precomp, conn, W, b):
    src, dst = ei[0], ei[1]
    n = x.shape[0]
    xj = x[src]
    cn = conn / (jnp.linalg.norm(conn, axis=-1, keepdims=True) + 1e-8)
    m_in = x.shape[1]
    e = cn.shape[0]
    rots = [jnp.stack([jnp.ones((e,), dtype=x.dtype),
                       jnp.zeros((e,), dtype=x.dtype)], axis=-1)]
    for _ in range(1, m_in):
        rots.append(_cmul(rots[-1], cn))
    rot = jnp.stack(rots, axis=1)
    xj = _cmul(xj, rot[:, :, None, :])
    aggs = []
    for r in range(precomp.shape[1]):
        contrib = _cmul(xj, precomp[:, r, None, None, :])
        aggs.append(jax.ops.segment_sum(contrib, dst, num_segments=n))
    agg = jnp.stack(aggs, axis=1)
    out = jnp.einsum('nrmci,orcd->nodi', agg, W)
    out = out.at[..., 0].add(b)
    return out


def _res_block(x, ei, pre, conn, p, name):
    h = _harmonic_conv(x, ei, pre, conn, p[name + '_W1'], p[name + '_b1'])
    h = _c_nonlin(h, p[name + '_nb1'])
    h = _harmonic_conv(h, ei, pre, conn, p[name + '_W2'], p[name + '_b2'])
    if name + '_Wres' in p:
        res = jnp.einsum('nmci,cd->nmdi', x, p[name + '_Wres'])
    else:
        res = x
    return _c_nonlin(h + res, p[name + '_nb2'])


def _pool_fn(x, idx, pconn):
    pc = pconn / (jnp.linalg.norm(pconn, axis=-1, keepdims=True) + 1e-8)
    return _cmul(x[idx], pc[:, None, None, :])


def _head_body(x_ref, w_ref, b_ref, o_ref):
    w = w_ref[...]
    b = b_ref[...]
    acc = None
    for m in range(2):
        xr = x_ref[:, (m * 2 + 0) * 32:(m * 2 + 1) * 32]
        xi = x_ref[:, (m * 2 + 1) * 32:(m * 2 + 2) * 32]
        yr = jnp.dot(xr, w, preferred_element_type=jnp.float32)
        yi = jnp.dot(xi, w, preferred_element_type=jnp.float32)
        sq = yr * yr + yi * yi
        mag = jnp.sqrt(sq + 1e-12)
        scale = jax.nn.relu(mag + b) / (mag + 1e-6)
        mag2 = jnp.sqrt(scale * scale * sq + 1e-12)
        acc = mag2 if acc is None else acc + mag2
    o_ref[...] = acc


@functools.partial(jax.jit, static_argnums=())
def _head(x_planar, w_pad, b_pad):
    n, _ = x_planar.shape
    jp = w_pad.shape[1]
    nb = 1000
    grid = (n // nb,)
    return pl.pallas_call(
        _head_body,
        grid=grid,
        in_specs=[
            pl.BlockSpec((nb, 128), lambda i: (i, 0)),
            pl.BlockSpec((32, jp), lambda i: (0, 0)),
            pl.BlockSpec((1, jp), lambda i: (0, 0)),
        ],
        out_specs=pl.BlockSpec((nb, jp), lambda i: (i, 0)),
        out_shape=jax.ShapeDtypeStruct((n, jp), jnp.float32),
    )(x_planar, w_pad, b_pad)


def _forward(p):
    pos = p['pos']
    x = jnp.stack([pos, jnp.zeros_like(pos)], axis=-1)[:, None, :, :]
    a0 = (p['edge_index0'], p['precomp0'], p['connection0'])
    a1 = (p['edge_index1'], p['precomp1'], p['connection1'])
    a2 = (p['edge_index2'], p['precomp2'], p['connection2'])
    a3 = (p['edge_index3'], p['precomp3'], p['connection3'])
    x = _res_block(x, a0[0], a0[1], a0[2], p, 'b01')
    x = _res_block(x, a0[0], a0[1], a0[2], p, 'b11')
    xp1 = _res_block(x, a0[0], a0[1], a0[2], p, 'b12')
    x = _pool_fn(xp1, p['pool_idx1'], p['pool_conn1'])
    x = _res_block(x, a1[0], a1[1], a1[2], p, 'b21')
    xp2 = _res_block(x, a1[0], a1[1], a1[2], p, 'b22')
    x = _pool_fn(xp2, p['pool_idx2'], p['pool_conn2'])
    x = _res_block(x, a2[0], a2[1], a2[2], p, 'b31')
    xp3 = _res_block(x, a2[0], a2[1], a2[2], p, 'b32')
    x = _pool_fn(xp3, p['pool_idx3'], p['pool_conn3'])
    x = _res_block(x, a3[0], a3[1], a3[2], p, 'b41')
    x = _res_block(x, a3[0], a3[1], a3[2], p, 'b42')
    x = _res_block(x, a3[0], a3[1], a3[2], p, 'b51')
    x = _res_block(x, a3[0], a3[1], a3[2], p, 'b52')
    x = x[p['unpool3']]
    x = jnp.concatenate([x, xp3], axis=2)
    x = _res_block(x, a2[0], a2[1], a2[2], p, 'b61')
    x = _res_block(x, a2[0], a2[1], a2[2], p, 'b62')
    x = x[p['unpool2']]
    x = jnp.concatenate([x, xp2], axis=2)
    x = _res_block(x, a1[0], a1[1], a1[2], p, 'b71')
    x = _res_block(x, a1[0], a1[1], a1[2], p, 'b72')
    x = x[p['unpool1']]
    x = jnp.concatenate([x, xp1], axis=2)
    x = _res_block(x, a0[0], a0[1], a0[2], p, 'b81')
    x = _res_block(x, a0[0], a0[1], a0[2], p, 'b82')

    n = x.shape[0]
    xp = jnp.transpose(x, (0, 1, 3, 2)).reshape(n, 128)
    jp = 384
    w_pad = jnp.zeros((32, jp), jnp.float32).at[:, :300].set(p['lin1_W'])
    b_pad = jnp.zeros((1, jp), jnp.float32).at[0, :300].set(p['nonlin1_b'])
    out = _head(xp, w_pad, b_pad)[:, :300]
    return out[None], pos[None]


def kernel(pos, edge_index0, precomp0, connection0, edge_index1, precomp1, connection1, edge_index2, precomp2, connection2, edge_index3, precomp3, connection3, pool_idx1, pool_conn1, pool_idx2, pool_conn2, pool_idx3, pool_conn3, unpool3, unpool2, unpool1, b01_W1, b01_b1, b01_nb1, b01_W2, b01_b2, b01_nb2, b01_Wres, b11_W1, b11_b1, b11_nb1, b11_W2, b11_b2, b11_nb2, b11_Wres, b12_W1, b12_b1, b12_nb1, b12_W2, b12_b2, b12_nb2, b21_W1, b21_b1, b21_nb1, b21_W2, b21_b2, b21_nb2, b21_Wres, b22_W1, b22_b1, b22_nb1, b22_W2, b22_b2, b22_nb2, b31_W1, b31_b1, b31_nb1, b31_W2, b31_b2, b31_nb2, b32_W1, b32_b1, b32_nb1, b32_W2, b32_b2, b32_nb2, b41_W1, b41_b1, b41_nb1, b41_W2, b41_b2, b41_nb2, b42_W1, b42_b1, b42_nb1, b42_W2, b42_b2, b42_nb2, b51_W1, b51_b1, b51_nb1, b51_W2, b51_b2, b51_nb2, b52_W1, b52_b1, b52_nb1, b52_W2, b52_b2, b52_nb2, b61_W1, b61_b1, b61_nb1, b61_W2, b61_b2, b61_nb2, b61_Wres, b62_W1, b62_b1, b62_nb1, b62_W2, b62_b2, b62_nb2, b71_W1, b71_b1, b71_nb1, b71_W2, b71_b2, b71_nb2, b71_Wres, b72_W1, b72_b1, b72_nb1, b72_W2, b72_b2, b72_nb2, b81_W1, b81_b1, b81_nb1, b81_W2, b81_b2, b81_nb2, b81_Wres, b82_W1, b82_b1, b82_nb1, b82_W2, b82_b2, b82_nb2, lin1_W, nonlin1_b):
    return _forward(dict(locals()))


# speedup vs baseline: 27.1391x; 27.1369x over previous
import functools

import jax
import jax.numpy as jnp
from jax import lax
from jax.experimental import pallas as pl
from jax.experimental.pallas import tpu as pltpu
from jax.experimental.pallas import tpu_sc as plsc

NC = 2
NS = 16
NW = NC * NS

@functools.lru_cache(maxsize=None)
def _mesh():
    return plsc.VectorSubcoreMesh(core_axis_name="c", subcore_axis_name="s")


@functools.lru_cache(maxsize=None)
def _make_gather(ntab, epad, k, cb, t):
    def body(tab, idx_hbm, out, idx_v, rows_v, sem):
        cid = lax.axis_index("c")
        sid = lax.axis_index("s")
        wid = sid * NC + cid

        def step(it, carry):
            base = (wid * t + it) * cb
            pltpu.sync_copy(idx_hbm.at[pl.ds(base, cb)], idx_v)
            pltpu.async_copy(tab.at[idx_v], rows_v, sem).wait()
            pltpu.sync_copy(rows_v, out.at[pl.ds(base, cb)])
            return carry

        lax.fori_loop(0, t, step, 0)

    return pl.kernel(
        body,
        out_type=jax.ShapeDtypeStruct((epad, k), jnp.float32),
        mesh=_mesh(),
        compiler_params=pltpu.CompilerParams(use_tc_tiling_on_sc=False),
        scratch_types=[
            pltpu.VMEM((cb,), jnp.int32),
            pltpu.VMEM((cb, k), jnp.float32),
            pltpu.SemaphoreType.DMA,
        ],
    )


def _sc_gather(tab, idx_pad, cb, t):
    ntab, k = tab.shape
    epad = idx_pad.shape[0]
    assert epad == NW * cb * t
    return _make_gather(ntab, epad, k, cb, t)(tab, idx_pad)


@functools.lru_cache(maxsize=None)
def _make_scatter(epad, k, nh, cb, t):
    zr = (nh + 16) // 16
    wr = nh // 16

    def body(msg_hbm, dst_hbm, zrow_hbm, out_hbm, idx_v, rows_v, z_v, acc):
        cid = lax.axis_index("c")
        sid = lax.axis_index("s")

        pltpu.sync_copy(zrow_hbm, z_v)
        zfull = zr // 16

        def zstep(q, carry):
            pltpu.sync_copy(z_v, acc.at[pl.ds(sid * zr + q * 16, 16)])
            return carry

        lax.fori_loop(0, zfull, zstep, 0)
        zrem = zr - zfull * 16
        if zrem:
            pltpu.sync_copy(z_v.at[pl.ds(0, zrem)],
                            acc.at[pl.ds(sid * zr + zfull * 16, zrem)])
        plsc.subcore_barrier()

        def step(it, carry):
            base = (sid * t + it) * cb
            pltpu.sync_copy(dst_hbm.at[pl.ds(base, cb)], idx_v)
            for j in range(cb // 16):
                v = idx_v[pl.ds(j * 16, 16)]
                lv = v - cid * nh
                ok = (lv >= 0) & (lv < nh)
                idx_v[pl.ds(j * 16, 16)] = jnp.where(ok, lv, nh)
            pltpu.sync_copy(msg_hbm.at[pl.ds(base, cb)], rows_v)
            pltpu.sync_copy(rows_v, acc.at[idx_v], add=True)
            return carry

        lax.fori_loop(0, t, step, 0)
        plsc.subcore_barrier()

        pltpu.sync_copy(acc.at[pl.ds(sid * wr, wr)],
                        out_hbm.at[pl.ds(cid * nh + sid * wr, wr)])

    return pl.kernel(
        body,
        out_type=jax.ShapeDtypeStruct((2 * nh, k), jnp.float32),
        mesh=_mesh(),
        compiler_params=pltpu.CompilerParams(use_tc_tiling_on_sc=False),
        scratch_types=[
            pltpu.VMEM((cb,), jnp.int32),
            pltpu.VMEM((cb, k), jnp.float32),
            pltpu.VMEM((16, k), jnp.float32),
            pltpu.VMEM_SHARED((nh + 16, k), jnp.float32),
        ],
    )


def _sc_scatter(msg, dst_pad, npad, cb, t):
    epad, k = msg.shape
    nh = npad // 2
    assert epad == NS * cb * t
    zrow = jnp.zeros((16, k), jnp.float32)
    return _make_scatter(epad, k, nh, cb, t)(msg, dst_pad, zrow)


@functools.lru_cache(maxsize=None)
def _make_msg(m_in, cpad, epad, eb):
    def body(xj_ref, q_ref, o_ref):
        q = q_ref[...]
        xs = [[xj_ref[:, (m * 2 + i) * cpad:(m * 2 + i + 1) * cpad]
               for i in range(2)] for m in range(m_in)]
        for r in range(2):
            re = None
            im = None
            for m in range(m_in):
                qr = q[:, (r * 2 + m) * 2 + 0][:, None]
                qi = q[:, (r * 2 + m) * 2 + 1][:, None]
                tr = qr * xs[m][0] - qi * xs[m][1]
                ti = qr * xs[m][1] + qi * xs[m][0]
                re = tr if re is None else re + tr
                im = ti if im is None else im + ti
            o_ref[:, (r * 2 + 0) * cpad:(r * 2 + 1) * cpad] = re
            o_ref[:, (r * 2 + 1) * cpad:(r * 2 + 2) * cpad] = im

    kx = m_in * 2 * cpad
    return pl.pallas_call(
        body,
        grid=(epad // eb,),
        in_specs=[
            pl.BlockSpec((eb, kx), lambda e: (e, 0)),
            pl.BlockSpec((eb, 8), lambda e: (e, 0)),
        ],
        out_specs=pl.BlockSpec((eb, 4 * cpad), lambda e: (e, 0)),
        out_shape=jax.ShapeDtypeStruct((epad, 4 * cpad), jnp.float32),
    )


def _tc_msg(xj, q8, m_in, cpad):
    epad = xj.shape[0]
    return _make_msg(m_in, cpad, epad, 512)(xj, q8)


@functools.lru_cache(maxsize=None)
def _make_conv_out(cpad, dpad, cres, has_res, npad, nb_rows, m_res=2):
    def body(*refs):
        if has_res:
            a_ref, w_ref, b_ref, nbias_ref, res_ref, wres_ref, o_ref = refs
        else:
            a_ref, w_ref, b_ref, nbias_ref, o_ref = refs
        w = w_ref[...]
        y = [None, None]
        for i in range(2):
            acc = jnp.broadcast_to(b_ref[...], (nb_rows, 2 * dpad)) if i == 0 \
                else jnp.zeros((nb_rows, 2 * dpad), jnp.float32)
            for r in range(2):
                a = a_ref[:, (r * 2 + i) * cpad:(r * 2 + i + 1) * cpad]
                acc = acc + jnp.dot(a, w[r * cpad:(r + 1) * cpad, :],
                                    preferred_element_type=jnp.float32)
            y[i] = acc
        if has_res:
            wres = wres_ref[...]
            for i in range(2):
                parts = []
                for o in range(2):
                    oe = min(o, m_res - 1)
                    rm = res_ref[:, (oe * 2 + i) * cres:(oe * 2 + i + 1) * cres]
                    parts.append(jnp.dot(rm, wres,
                                         preferred_element_type=jnp.float32))
                y[i] = y[i] + jnp.concatenate(parts, axis=1)
        sq = y[0] * y[0] + y[1] * y[1]
        mag = jnp.sqrt(sq + 1e-12)
        scale = jax.nn.relu(mag + nbias_ref[...]) / (mag + 1e-6)
        for o in range(2):
            for i in range(2):
                o_ref[:, (o * 2 + i) * dpad:(o * 2 + i + 1) * dpad] = \
                    (y[i] * scale)[:, o * dpad:(o + 1) * dpad]

    in_specs = [
        pl.BlockSpec((nb_rows, 4 * cpad), lambda n: (n, 0)),
        pl.BlockSpec((2 * cpad, 2 * dpad), lambda n: (0, 0)),
        pl.BlockSpec((1, 2 * dpad), lambda n: (0, 0)),
        pl.BlockSpec((1, 2 * dpad), lambda n: (0, 0)),
    ]
    if has_res:
        in_specs += [
            pl.BlockSpec((nb_rows, 2 * m_res * cres), lambda n: (n, 0)),
            pl.BlockSpec((cres, dpad), lambda n: (0, 0)),
        ]
    return pl.pallas_call(
        body,
        grid=(npad // nb_rows,),
        in_specs=in_specs,
        out_specs=pl.BlockSpec((nb_rows, 4 * dpad), lambda n: (n, 0)),
        out_shape=jax.ShapeDtypeStruct((npad, 4 * dpad), jnp.float32),
    )


def _tc_conv_out(agg, w, b2, nb2, res=None, wres=None):
    npad, ka = agg.shape
    cpad = ka // 4
    dpad = w.shape[1] // 2
    if res is not None:
        cres = wres.shape[0]
        m_res = res.shape[1] // (2 * cres)
        return _make_conv_out(cpad, dpad, cres, True, npad, 256, m_res)(
            agg, w, b2, nb2, res, wres)
    return _make_conv_out(cpad, dpad, 0, False, npad, 256)(agg, w, b2, nb2)


def _head_body(x_ref, w_ref, b_ref, o_ref):
    w = w_ref[...]
    b = b_ref[...]
    acc = None
    for m in range(2):
        xr = x_ref[:, (m * 2 + 0) * 32:(m * 2 + 1) * 32]
        xi = x_ref[:, (m * 2 + 1) * 32:(m * 2 + 2) * 32]
        yr = jnp.dot(xr, w, preferred_element_type=jnp.float32)
        yi = jnp.dot(xi, w, preferred_element_type=jnp.float32)
        sq = yr * yr + yi * yi
        mag = jnp.sqrt(sq + 1e-12)
        scale = jax.nn.relu(mag + b) / (mag + 1e-6)
        mag2 = jnp.sqrt(scale * scale * sq + 1e-12)
        acc = mag2 if acc is None else acc + mag2
    o_ref[...] = acc


def _head(x_planar, w_pad, b_pad):
    n, _ = x_planar.shape
    jp = w_pad.shape[1]
    nb = 512
    return pl.pallas_call(
        _head_body,
        grid=(n // nb,),
        in_specs=[
            pl.BlockSpec((nb, 128), lambda i: (i, 0)),
            pl.BlockSpec((32, jp), lambda i: (0, 0)),
            pl.BlockSpec((1, jp), lambda i: (0, 0)),
        ],
        out_specs=pl.BlockSpec((nb, jp), lambda i: (i, 0)),
        out_shape=jax.ShapeDtypeStruct((n, jp), jnp.float32),
    )(x_planar, w_pad, b_pad)


def _pad_rows(a, rows, fill=0):
    pad = rows - a.shape[0]
    if pad == 0:
        return a
    return jnp.concatenate(
        [a, jnp.full((pad,) + a.shape[1:], fill, a.dtype)], axis=0)


def _prep_w(w, cin, cout, cpad, dpad):
    wp = jnp.zeros((2, 2, cpad, dpad), jnp.float32)
    wp = wp.at[:, :, :cin, :cout].set(w)
    return wp.transpose(1, 2, 0, 3).reshape(2 * cpad, 2 * dpad)


def _prep_b(b, cout, dpad):
    bp = jnp.zeros((dpad,), jnp.float32).at[:cout].set(b)
    return jnp.concatenate([bp, bp])[None]


def _norm2(v):
    return v / (jnp.linalg.norm(v, axis=-1, keepdims=True) + 1e-8)


def _rotate_rows(tab, pc, cpad):
    n = tab.shape[0]
    x = tab.reshape(n, 2, 2, cpad)
    pr = pc[:, 0][:, None, None]
    pi = pc[:, 1][:, None, None]
    re = x[:, :, 0] * pr - x[:, :, 1] * pi
    im = x[:, :, 1] * pr + x[:, :, 0] * pi
    return jnp.stack([re, im], axis=2).reshape(n, 4 * cpad)


_LEVEL = {
    0: dict(n=10000, npad=10240, e=160000, epad=161792, gcb=64, gt=79,
            scb=64, st=158),
    1: dict(n=5000, npad=5120, e=80000, epad=81920, gcb=64, gt=40,
            scb=64, st=80),
    2: dict(n=2500, npad=2560, e=40000, epad=40960, gcb=64, gt=20,
            scb=64, st=40),
    3: dict(n=1250, npad=1280, e=20000, epad=20480, gcb=64, gt=10,
            scb=64, st=20),
}
_IDXG = {5120: (32, 5), 2560: (16, 5), 1280: (8, 5), 10240: (64, 5)}


def _forward(p):
    pos = p['pos']

    levels = {}
    for s in range(4):
        lv = dict(_LEVEL[s])
        ei = p['edge_index%d' % s]
        src = _pad_rows(ei[0].astype(jnp.int32), lv['epad'], 0)
        dst = _pad_rows(ei[1].astype(jnp.int32), lv['epad'], -1)
        cn = _norm2(p['connection%d' % s])
        pre = p['precomp%d' % s]
        q = jnp.zeros((lv['e'], 2, 2, 2), jnp.float32)
        q = q.at[:, :, 0, :].set(pre)
        qr = pre[:, :, 0] * cn[:, None, 0] - pre[:, :, 1] * cn[:, None, 1]
        qi = pre[:, :, 0] * cn[:, None, 1] + pre[:, :, 1] * cn[:, None, 0]
        q = q.at[:, :, 1, 0].set(qr).at[:, :, 1, 1].set(qi)
        lv['src'] = src
        lv['dst'] = dst
        lv['q8'] = _pad_rows(q.reshape(lv['e'], 8), lv['epad'])
        levels[s] = lv

    def conv(x_tab, s, m_in, cin, cout, wkey, bkey, nbkey):
        lv = levels[s]
        cpad = max(8, cin)
        dpad = cout
        xj = _sc_gather(x_tab, lv['src'], lv['gcb'], lv['gt'])
        msg = _tc_msg(xj, lv['q8'], m_in, cpad)
        agg = _sc_scatter(msg, lv['dst'], lv['npad'], lv['scb'], lv['st'])
        w = _prep_w(p[wkey], cin, cout, cpad, dpad)
        return agg, w, _prep_b(p[bkey], cout, dpad), _prep_b(p[nbkey], cout, dpad)

    def res_block(x_tab, s, name, cin, cout, m_in=2):
        cpad_in = max(8, cin)
        agg, w1, b1, nb1 = conv(x_tab, s, m_in, cin, cout,
                                name + '_W1', name + '_b1', name + '_nb1')
        h = _tc_conv_out(agg, w1, b1, nb1)
        agg2, w2, b2, nb2 = conv(h, s, 2, cout, cout,
                                 name + '_W2', name + '_b2', name + '_nb2')
        if name + '_Wres' in p:
            wres = jnp.zeros((cpad_in, cout), jnp.float32)
            wres = wres.at[:cin, :].set(p[name + '_Wres'])
        else:
            wres = jnp.eye(cpad_in, cout, dtype=jnp.float32)
        return _tc_conv_out(agg2, w2, b2, nb2, res=x_tab, wres=wres)

    def pool(x_tab, l, ncoarse_pad):
        cb, t = _IDXG[ncoarse_pad]
        idx = _pad_rows(p['pool_idx%d' % l].astype(jnp.int32), ncoarse_pad, 0)
        g = _sc_gather(x_tab, idx, cb, t)
        pc = _pad_rows(_norm2(p['pool_conn%d' % l]), ncoarse_pad, 0)
        return _rotate_rows(g, pc, x_tab.shape[1] // 4)

    def unpool(x_coarse, xp_tab, idx_raw, nfine_pad):
        cb, t = _IDXG[nfine_pad]
        idx = _pad_rows(idx_raw.astype(jnp.int32), nfine_pad, 0)
        up = _sc_gather(x_coarse, idx, cb, t)
        n = nfine_pad
        ca = up.shape[1] // 4
        cb2 = xp_tab.shape[1] // 4
        cat = jnp.concatenate([up.reshape(n, 4, ca), xp_tab.reshape(n, 4, cb2)],
                              axis=2)
        return cat.reshape(n, 4 * (ca + cb2))

    x0 = jnp.zeros((_LEVEL[0]['npad'], 16), jnp.float32)
    x0 = x0.at[:10000, 0:3].set(pos)

    x = res_block(x0, 0, 'b01', 3, 16, m_in=1)
    x = res_block(x, 0, 'b11', 16, 32)
    xp1 = res_block(x, 0, 'b12', 32, 32)
    x = pool(xp1, 1, 5120)
    x = res_block(x, 1, 'b21', 32, 64)
    xp2 = res_block(x, 1, 'b22', 64, 64)
    x = pool(xp2, 2, 2560)
    x = res_block(x, 2, 'b31', 64, 64)
    xp3 = res_block(x, 2, 'b32', 64, 64)
    x = pool(xp3, 3, 1280)
    x = res_block(x, 3, 'b41', 64, 64)
    x = res_block(x, 3, 'b42', 64, 64)
    x = res_block(x, 3, 'b51', 64, 64)
    x = res_block(x, 3, 'b52', 64, 64)
    x = unpool(x, xp3, p['unpool3'], 2560)
    x = res_block(x, 2, 'b61', 128, 64)
    x = res_block(x, 2, 'b62', 64, 64)
    x = unpool(x, xp2, p['unpool2'], 5120)
    x = res_block(x, 1, 'b71', 128, 32)
    x = res_block(x, 1, 'b72', 32, 32)
    x = unpool(x, xp1, p['unpool1'], 10240)
    x = res_block(x, 0, 'b81', 64, 32)
    x = res_block(x, 0, 'b82', 32, 32)

    jp = 384
    w_pad = jnp.zeros((32, jp), jnp.float32).at[:, :300].set(p['lin1_W'])
    b_pad = jnp.zeros((1, jp), jnp.float32).at[0, :300].set(p['nonlin1_b'])
    out = _head(x, w_pad, b_pad)[:10000, :300]
    return out[None], pos[None]


def kernel(pos, edge_index0, precomp0, connection0, edge_index1, precomp1, connection1, edge_index2, precomp2, connection2, edge_index3, precomp3, connection3, pool_idx1, pool_conn1, pool_idx2, pool_conn2, pool_idx3, pool_conn3, unpool3, unpool2, unpool1, b01_W1, b01_b1, b01_nb1, b01_W2, b01_b2, b01_nb2, b01_Wres, b11_W1, b11_b1, b11_nb1, b11_W2, b11_b2, b11_nb2, b11_Wres, b12_W1, b12_b1, b12_nb1, b12_W2, b12_b2, b12_nb2, b21_W1, b21_b1, b21_nb1, b21_W2, b21_b2, b21_nb2, b21_Wres, b22_W1, b22_b1, b22_nb1, b22_W2, b22_b2, b22_nb2, b31_W1, b31_b1, b31_nb1, b31_W2, b31_b2, b31_nb2, b32_W1, b32_b1, b32_nb1, b32_W2, b32_b2, b32_nb2, b41_W1, b41_b1, b41_nb1, b41_W2, b41_b2, b41_nb2, b42_W1, b42_b1, b42_nb1, b42_W2, b42_b2, b42_nb2, b51_W1, b51_b1, b51_nb1, b51_W2, b51_b2, b51_nb2, b52_W1, b52_b1, b52_nb1, b52_W2, b52_b2, b52_nb2, b61_W1, b61_b1, b61_nb1, b61_W2, b61_b2, b61_nb2, b61_Wres, b62_W1, b62_b1, b62_nb1, b62_W2, b62_b2, b62_nb2, b71_W1, b71_b1, b71_nb1, b71_W2, b71_b2, b71_nb2, b71_Wres, b72_W1, b72_b1, b72_nb1, b72_W2, b72_b2, b72_nb2, b81_W1, b81_b1, b81_nb1, b81_W2, b81_b2, b81_nb2, b81_Wres, b82_W1, b82_b1, b82_nb1, b82_W2, b82_b2, b82_nb2, lin1_W, nonlin1_b):
    return _forward(dict(locals()))


# speedup vs baseline: 28.2982x; 1.0427x over previous
import functools

import jax
import jax.numpy as jnp
from jax import lax
from jax.experimental import pallas as pl
from jax.experimental.pallas import tpu as pltpu
from jax.experimental.pallas import tpu_sc as plsc

NC = 2
NS = 16
NW = NC * NS

@functools.lru_cache(maxsize=None)
def _mesh():
    return plsc.VectorSubcoreMesh(core_axis_name="c", subcore_axis_name="s")


@functools.lru_cache(maxsize=None)
def _make_gather(ntab, epad, k, cb, t):
    def body(tab, idx_hbm, out, idx_v, rows_v, sem):
        cid = lax.axis_index("c")
        sid = lax.axis_index("s")
        wid = sid * NC + cid

        def step(it, carry):
            base = (wid * t + it) * cb
            pltpu.sync_copy(idx_hbm.at[pl.ds(base, cb)], idx_v)
            pltpu.async_copy(tab.at[idx_v], rows_v, sem).wait()
            pltpu.sync_copy(rows_v, out.at[pl.ds(base, cb)])
            return carry

        lax.fori_loop(0, t, step, 0)

    return pl.kernel(
        body,
        out_type=jax.ShapeDtypeStruct((epad, k), jnp.float32),
        mesh=_mesh(),
        compiler_params=pltpu.CompilerParams(use_tc_tiling_on_sc=False),
        scratch_types=[
            pltpu.VMEM((cb,), jnp.int32),
            pltpu.VMEM((cb, k), jnp.float32),
            pltpu.SemaphoreType.DMA,
        ],
    )


def _sc_gather(tab, idx_pad, cb, t):
    ntab, k = tab.shape
    epad = idx_pad.shape[0]
    assert epad == NW * cb * t
    return _make_gather(ntab, epad, k, cb, t)(tab, idx_pad)


@functools.lru_cache(maxsize=None)
def _make_scatter(epad, kfull, k, coff, nh, cb, t):
    zr = (nh + 16) // 16
    wr = nh // 16

    def body(msg_hbm, dst_hbm, zrow_hbm, out_hbm, idx_v, rows_v, z_v, acc):
        cid = lax.axis_index("c")
        sid = lax.axis_index("s")

        pltpu.sync_copy(zrow_hbm, z_v)
        zfull = zr // 16

        def zstep(q, carry):
            pltpu.sync_copy(z_v, acc.at[pl.ds(sid * zr + q * 16, 16)])
            return carry

        lax.fori_loop(0, zfull, zstep, 0)
        zrem = zr - zfull * 16
        if zrem:
            pltpu.sync_copy(z_v.at[pl.ds(0, zrem)],
                            acc.at[pl.ds(sid * zr + zfull * 16, zrem)])
        plsc.subcore_barrier()

        def step(it, carry):
            base = (sid * t + it) * cb
            pltpu.sync_copy(dst_hbm.at[pl.ds(base, cb)], idx_v)
            for j in range(cb // 16):
                v = idx_v[pl.ds(j * 16, 16)]
                lv = v - cid * nh
                ok = (lv >= 0) & (lv < nh)
                idx_v[pl.ds(j * 16, 16)] = jnp.where(ok, lv, nh)
            pltpu.sync_copy(msg_hbm.at[pl.ds(base, cb), pl.ds(coff, k)],
                            rows_v)
            pltpu.sync_copy(rows_v, acc.at[idx_v], add=True)
            return carry

        lax.fori_loop(0, t, step, 0)
        plsc.subcore_barrier()

        pltpu.sync_copy(acc.at[pl.ds(sid * wr, wr)],
                        out_hbm.at[pl.ds(cid * nh + sid * wr, wr)])

    return pl.kernel(
        body,
        out_type=jax.ShapeDtypeStruct((2 * nh, k), jnp.float32),
        mesh=_mesh(),
        compiler_params=pltpu.CompilerParams(use_tc_tiling_on_sc=False),
        scratch_types=[
            pltpu.VMEM((cb,), jnp.int32),
            pltpu.VMEM((cb, k), jnp.float32),
            pltpu.VMEM((16, k), jnp.float32),
            pltpu.VMEM_SHARED((nh + 16, k), jnp.float32),
        ],
    )


def _sc_scatter(msg, dst_pad, npad, cb, t):
    epad, k = msg.shape
    nh = npad // 2
    assert epad == NS * cb * t
    nsplit = 1
    while (nh + 16) * (k // nsplit) * 4 > 3_000_000:
        nsplit *= 2
    kk = k // nsplit
    zrow = jnp.zeros((16, kk), jnp.float32)
    parts = [_make_scatter(epad, k, kk, si * kk, nh, cb, t)(msg, dst_pad, zrow)
             for si in range(nsplit)]
    if nsplit == 1:
        return parts[0]
    return jnp.concatenate(parts, axis=1)


@functools.lru_cache(maxsize=None)
def _make_msg(m_in, cpad, epad, eb):
    def body(xj_ref, q_ref, o_ref):
        q = q_ref[...]
        xs = [[xj_ref[:, (m * 2 + i) * cpad:(m * 2 + i + 1) * cpad]
               for i in range(2)] for m in range(m_in)]
        for r in range(2):
            re = None
            im = None
            for m in range(m_in):
                qr = q[:, (r * 2 + m) * 2 + 0][:, None]
                qi = q[:, (r * 2 + m) * 2 + 1][:, None]
                tr = qr * xs[m][0] - qi * xs[m][1]
                ti = qr * xs[m][1] + qi * xs[m][0]
                re = tr if re is None else re + tr
                im = ti if im is None else im + ti
            o_ref[:, (r * 2 + 0) * cpad:(r * 2 + 1) * cpad] = re
            o_ref[:, (r * 2 + 1) * cpad:(r * 2 + 2) * cpad] = im

    kx = m_in * 2 * cpad
    return pl.pallas_call(
        body,
        grid=(epad // eb,),
        in_specs=[
            pl.BlockSpec((eb, kx), lambda e: (e, 0)),
            pl.BlockSpec((eb, 8), lambda e: (e, 0)),
        ],
        out_specs=pl.BlockSpec((eb, 4 * cpad), lambda e: (e, 0)),
        out_shape=jax.ShapeDtypeStruct((epad, 4 * cpad), jnp.float32),
    )


def _tc_msg(xj, q8, m_in, cpad):
    epad = xj.shape[0]
    return _make_msg(m_in, cpad, epad, 512)(xj, q8)


@functools.lru_cache(maxsize=None)
def _make_conv_out(cpad, dpad, cres, has_res, npad, nb_rows, m_res=2):
    def body(*refs):
        if has_res:
            a_ref, w_ref, b_ref, nbias_ref, res_ref, wres_ref, o_ref = refs
        else:
            a_ref, w_ref, b_ref, nbias_ref, o_ref = refs
        w = w_ref[...]
        y = [None, None]
        for i in range(2):
            acc = jnp.broadcast_to(b_ref[...], (nb_rows, 2 * dpad)) if i == 0 \
                else jnp.zeros((nb_rows, 2 * dpad), jnp.float32)
            for r in range(2):
                a = a_ref[:, (r * 2 + i) * cpad:(r * 2 + i + 1) * cpad]
                acc = acc + jnp.dot(a, w[r * cpad:(r + 1) * cpad, :],
                                    preferred_element_type=jnp.float32)
            y[i] = acc
        if has_res:
            wres = wres_ref[...]
            for i in range(2):
                parts = []
                for o in range(2):
                    oe = min(o, m_res - 1)
                    rm = res_ref[:, (oe * 2 + i) * cres:(oe * 2 + i + 1) * cres]
                    parts.append(jnp.dot(rm, wres,
                                         preferred_element_type=jnp.float32))
                y[i] = y[i] + jnp.concatenate(parts, axis=1)
        sq = y[0] * y[0] + y[1] * y[1]
        mag = jnp.sqrt(sq + 1e-12)
        scale = jax.nn.relu(mag + nbias_ref[...]) / (mag + 1e-6)
        for o in range(2):
            for i in range(2):
                o_ref[:, (o * 2 + i) * dpad:(o * 2 + i + 1) * dpad] = \
                    (y[i] * scale)[:, o * dpad:(o + 1) * dpad]

    in_specs = [
        pl.BlockSpec((nb_rows, 4 * cpad), lambda n: (n, 0)),
        pl.BlockSpec((2 * cpad, 2 * dpad), lambda n: (0, 0)),
        pl.BlockSpec((1, 2 * dpad), lambda n: (0, 0)),
        pl.BlockSpec((1, 2 * dpad), lambda n: (0, 0)),
    ]
    if has_res:
        in_specs += [
            pl.BlockSpec((nb_rows, 2 * m_res * cres), lambda n: (n, 0)),
            pl.BlockSpec((cres, dpad), lambda n: (0, 0)),
        ]
    return pl.pallas_call(
        body,
        grid=(npad // nb_rows,),
        in_specs=in_specs,
        out_specs=pl.BlockSpec((nb_rows, 4 * dpad), lambda n: (n, 0)),
        out_shape=jax.ShapeDtypeStruct((npad, 4 * dpad), jnp.float32),
    )


def _tc_conv_out(agg, w, b2, nb2, res=None, wres=None):
    npad, ka = agg.shape
    cpad = ka // 4
    dpad = w.shape[1] // 2
    if res is not None:
        cres = wres.shape[0]
        m_res = res.shape[1] // (2 * cres)
        return _make_conv_out(cpad, dpad, cres, True, npad, 256, m_res)(
            agg, w, b2, nb2, res, wres)
    return _make_conv_out(cpad, dpad, 0, False, npad, 256)(agg, w, b2, nb2)


def _head_body(x_ref, w_ref, b_ref, o_ref):
    w = w_ref[...]
    b = b_ref[...]
    acc = None
    for m in range(2):
        xr = x_ref[:, (m * 2 + 0) * 32:(m * 2 + 1) * 32]
        xi = x_ref[:, (m * 2 + 1) * 32:(m * 2 + 2) * 32]
        yr = jnp.dot(xr, w, preferred_element_type=jnp.float32)
        yi = jnp.dot(xi, w, preferred_element_type=jnp.float32)
        sq = yr * yr + yi * yi
        mag = jnp.sqrt(sq + 1e-12)
        scale = jax.nn.relu(mag + b) / (mag + 1e-6)
        mag2 = jnp.sqrt(scale * scale * sq + 1e-12)
        acc = mag2 if acc is None else acc + mag2
    o_ref[...] = acc


def _head(x_planar, w_pad, b_pad):
    n, _ = x_planar.shape
    jp = w_pad.shape[1]
    nb = 512
    return pl.pallas_call(
        _head_body,
        grid=(n // nb,),
        in_specs=[
            pl.BlockSpec((nb, 128), lambda i: (i, 0)),
            pl.BlockSpec((32, jp), lambda i: (0, 0)),
            pl.BlockSpec((1, jp), lambda i: (0, 0)),
        ],
        out_specs=pl.BlockSpec((nb, jp), lambda i: (i, 0)),
        out_shape=jax.ShapeDtypeStruct((n, jp), jnp.float32),
    )(x_planar, w_pad, b_pad)


def _pad_rows(a, rows, fill=0):
    pad = rows - a.shape[0]
    if pad == 0:
        return a
    return jnp.concatenate(
        [a, jnp.full((pad,) + a.shape[1:], fill, a.dtype)], axis=0)


def _prep_w(w, cin, cout, cpad, dpad):
    wp = jnp.zeros((2, 2, cpad, dpad), jnp.float32)
    wp = wp.at[:, :, :cin, :cout].set(w)
    return wp.transpose(1, 2, 0, 3).reshape(2 * cpad, 2 * dpad)


def _prep_b(b, cout, dpad):
    bp = jnp.zeros((dpad,), jnp.float32).at[:cout].set(b)
    return jnp.concatenate([bp, bp])[None]


def _norm2(v):
    return v / (jnp.linalg.norm(v, axis=-1, keepdims=True) + 1e-8)


def _rotate_rows(tab, pc, cpad):
    n = tab.shape[0]
    x = tab.reshape(n, 2, 2, cpad)
    pr = pc[:, 0][:, None, None]
    pi = pc[:, 1][:, None, None]
    re = x[:, :, 0] * pr - x[:, :, 1] * pi
    im = x[:, :, 1] * pr + x[:, :, 0] * pi
    return jnp.stack([re, im], axis=2).reshape(n, 4 * cpad)


_LEVEL = {
    0: dict(n=10000, npad=10240, e=160000, epad=163840, gcb=128, gt=40,
            scb=128, st=80),
    1: dict(n=5000, npad=5120, e=80000, epad=81920, gcb=128, gt=20,
            scb=128, st=40),
    2: dict(n=2500, npad=2560, e=40000, epad=40960, gcb=128, gt=10,
            scb=128, st=20),
    3: dict(n=1250, npad=1280, e=20000, epad=20480, gcb=128, gt=5,
            scb=128, st=10),
}
_IDXG = {5120: (32, 5), 2560: (16, 5), 1280: (8, 5), 10240: (64, 5)}


def _forward(p):
    pos = p['pos']

    levels = {}
    for s in range(4):
        lv = dict(_LEVEL[s])
        ei = p['edge_index%d' % s]
        src = _pad_rows(ei[0].astype(jnp.int32), lv['epad'], 0)
        dst = _pad_rows(ei[1].astype(jnp.int32), lv['epad'], -1)
        cn = _norm2(p['connection%d' % s])
        pre = p['precomp%d' % s]
        q = jnp.zeros((lv['e'], 2, 2, 2), jnp.float32)
        q = q.at[:, :, 0, :].set(pre)
        qr = pre[:, :, 0] * cn[:, None, 0] - pre[:, :, 1] * cn[:, None, 1]
        qi = pre[:, :, 0] * cn[:, None, 1] + pre[:, :, 1] * cn[:, None, 0]
        q = q.at[:, :, 1, 0].set(qr).at[:, :, 1, 1].set(qi)
        lv['src'] = src
        lv['dst'] = dst
        lv['q8'] = _pad_rows(q.reshape(lv['e'], 8), lv['epad'])
        levels[s] = lv

    def conv(x_tab, s, m_in, cin, cout, wkey, bkey, nbkey):
        lv = levels[s]
        cpad = max(8, cin)
        dpad = cout
        xj = _sc_gather(x_tab, lv['src'], lv['gcb'], lv['gt'])
        msg = _tc_msg(xj, lv['q8'], m_in, cpad)
        agg = _sc_scatter(msg, lv['dst'], lv['npad'], lv['scb'], lv['st'])
        w = _prep_w(p[wkey], cin, cout, cpad, dpad)
        return agg, w, _prep_b(p[bkey], cout, dpad), _prep_b(p[nbkey], cout, dpad)

    def res_block(x_tab, s, name, cin, cout, m_in=2):
        cpad_in = max(8, cin)
        agg, w1, b1, nb1 = conv(x_tab, s, m_in, cin, cout,
                                name + '_W1', name + '_b1', name + '_nb1')
        h = _tc_conv_out(agg, w1, b1, nb1)
        agg2, w2, b2, nb2 = conv(h, s, 2, cout, cout,
                                 name + '_W2', name + '_b2', name + '_nb2')
        if name + '_Wres' in p:
            wres = jnp.zeros((cpad_in, cout), jnp.float32)
            wres = wres.at[:cin, :].set(p[name + '_Wres'])
        else:
            wres = jnp.eye(cpad_in, cout, dtype=jnp.float32)
        return _tc_conv_out(agg2, w2, b2, nb2, res=x_tab, wres=wres)

    def pool(x_tab, l, ncoarse_pad):
        cb, t = _IDXG[ncoarse_pad]
        idx = _pad_rows(p['pool_idx%d' % l].astype(jnp.int32), ncoarse_pad, 0)
        g = _sc_gather(x_tab, idx, cb, t)
        pc = _pad_rows(_norm2(p['pool_conn%d' % l]), ncoarse_pad, 0)
        return _rotate_rows(g, pc, x_tab.shape[1] // 4)

    def unpool(x_coarse, xp_tab, idx_raw, nfine_pad):
        cb, t = _IDXG[nfine_pad]
        idx = _pad_rows(idx_raw.astype(jnp.int32), nfine_pad, 0)
        up = _sc_gather(x_coarse, idx, cb, t)
        n = nfine_pad
        ca = up.shape[1] // 4
        cb2 = xp_tab.shape[1] // 4
        cat = jnp.concatenate([up.reshape(n, 4, ca), xp_tab.reshape(n, 4, cb2)],
                              axis=2)
        return cat.reshape(n, 4 * (ca + cb2))

    x0 = jnp.zeros((_LEVEL[0]['npad'], 16), jnp.float32)
    x0 = x0.at[:10000, 0:3].set(pos)

    x = res_block(x0, 0, 'b01', 3, 16, m_in=1)
    x = res_block(x, 0, 'b11', 16, 32)
    xp1 = res_block(x, 0, 'b12', 32, 32)
    x = pool(xp1, 1, 5120)
    x = res_block(x, 1, 'b21', 32, 64)
    xp2 = res_block(x, 1, 'b22', 64, 64)
    x = pool(xp2, 2, 2560)
    x = res_block(x, 2, 'b31', 64, 64)
    xp3 = res_block(x, 2, 'b32', 64, 64)
    x = pool(xp3, 3, 1280)
    x = res_block(x, 3, 'b41', 64, 64)
    x = res_block(x, 3, 'b42', 64, 64)
    x = res_block(x, 3, 'b51', 64, 64)
    x = res_block(x, 3, 'b52', 64, 64)
    x = unpool(x, xp3, p['unpool3'], 2560)
    x = res_block(x, 2, 'b61', 128, 64)
    x = res_block(x, 2, 'b62', 64, 64)
    x = unpool(x, xp2, p['unpool2'], 5120)
    x = res_block(x, 1, 'b71', 128, 32)
    x = res_block(x, 1, 'b72', 32, 32)
    x = unpool(x, xp1, p['unpool1'], 10240)
    x = res_block(x, 0, 'b81', 64, 32)
    x = res_block(x, 0, 'b82', 32, 32)

    jp = 384
    w_pad = jnp.zeros((32, jp), jnp.float32).at[:, :300].set(p['lin1_W'])
    b_pad = jnp.zeros((1, jp), jnp.float32).at[0, :300].set(p['nonlin1_b'])
    out = _head(x, w_pad, b_pad)[:10000, :300]
    return out[None], pos[None]


def kernel(pos, edge_index0, precomp0, connection0, edge_index1, precomp1, connection1, edge_index2, precomp2, connection2, edge_index3, precomp3, connection3, pool_idx1, pool_conn1, pool_idx2, pool_conn2, pool_idx3, pool_conn3, unpool3, unpool2, unpool1, b01_W1, b01_b1, b01_nb1, b01_W2, b01_b2, b01_nb2, b01_Wres, b11_W1, b11_b1, b11_nb1, b11_W2, b11_b2, b11_nb2, b11_Wres, b12_W1, b12_b1, b12_nb1, b12_W2, b12_b2, b12_nb2, b21_W1, b21_b1, b21_nb1, b21_W2, b21_b2, b21_nb2, b21_Wres, b22_W1, b22_b1, b22_nb1, b22_W2, b22_b2, b22_nb2, b31_W1, b31_b1, b31_nb1, b31_W2, b31_b2, b31_nb2, b32_W1, b32_b1, b32_nb1, b32_W2, b32_b2, b32_nb2, b41_W1, b41_b1, b41_nb1, b41_W2, b41_b2, b41_nb2, b42_W1, b42_b1, b42_nb1, b42_W2, b42_b2, b42_nb2, b51_W1, b51_b1, b51_nb1, b51_W2, b51_b2, b51_nb2, b52_W1, b52_b1, b52_nb1, b52_W2, b52_b2, b52_nb2, b61_W1, b61_b1, b61_nb1, b61_W2, b61_b2, b61_nb2, b61_Wres, b62_W1, b62_b1, b62_nb1, b62_W2, b62_b2, b62_nb2, b71_W1, b71_b1, b71_nb1, b71_W2, b71_b2, b71_nb2, b71_Wres, b72_W1, b72_b1, b72_nb1, b72_W2, b72_b2, b72_nb2, b81_W1, b81_b1, b81_nb1, b81_W2, b81_b2, b81_nb2, b81_Wres, b82_W1, b82_b1, b82_nb1, b82_W2, b82_b2, b82_nb2, lin1_W, nonlin1_b):
    return _forward(dict(locals()))


# speedup vs baseline: 29.6478x; 1.0477x over previous
import functools

import jax
import jax.numpy as jnp
from jax import lax
from jax.experimental import pallas as pl
from jax.experimental.pallas import tpu as pltpu
from jax.experimental.pallas import tpu_sc as plsc

NC = 2
NS = 16
NW = NC * NS

@functools.lru_cache(maxsize=None)
def _mesh():
    return plsc.VectorSubcoreMesh(core_axis_name="c", subcore_axis_name="s")


@functools.lru_cache(maxsize=None)
def _make_gather(ntab, epad, k, cb, t):
    assert t >= 2 and t % 2 == 0

    def body(tab, idx_hbm, out, idx_v, r0, r1, s0, s1):
        cid = lax.axis_index("c")
        sid = lax.axis_index("s")
        wid = sid * NC + cid
        tbase = wid * t * cb
        pltpu.sync_copy(idx_hbm.at[pl.ds(tbase, t * cb)], idx_v)

        def gat(it, buf, sem):
            return pltpu.async_copy(tab.at[idx_v.at[pl.ds(it * cb, cb)]],
                                    buf, sem)

        def wout(it, buf, sem):
            pltpu.make_async_copy(tab.at[idx_v.at[pl.ds(it * cb, cb)]],
                                  buf, sem).wait()
            pltpu.sync_copy(buf, out.at[pl.ds(tbase + it * cb, cb)])

        gat(0, r0, s0)
        gat(1, r1, s1)

        def step(kk, carry):
            t1 = 2 * kk
            wout(t1, r0, s0)
            gat(t1 + 2, r0, s0)
            wout(t1 + 1, r1, s1)
            gat(t1 + 3, r1, s1)
            return carry

        lax.fori_loop(0, (t - 2) // 2, step, 0)
        wout(t - 2, r0, s0)
        wout(t - 1, r1, s1)

    return pl.kernel(
        body,
        out_type=jax.ShapeDtypeStruct((epad, k), jnp.float32),
        mesh=_mesh(),
        compiler_params=pltpu.CompilerParams(use_tc_tiling_on_sc=False),
        scratch_types=[
            pltpu.VMEM((t * cb,), jnp.int32),
            pltpu.VMEM((cb, k), jnp.float32),
            pltpu.VMEM((cb, k), jnp.float32),
            pltpu.SemaphoreType.DMA,
            pltpu.SemaphoreType.DMA,
        ],
    )


def _sc_gather(tab, idx_pad, cb=None, t=None):
    ntab, k = tab.shape
    epad = idx_pad.shape[0]
    if cb is None:
        cb = 32 if k >= 512 else (64 if k >= 256 else 128)
        t = epad // (NW * cb)
    assert epad == NW * cb * t and t % 2 == 0, (epad, k, cb, t)
    return _make_gather(ntab, epad, k, cb, t)(tab, idx_pad)


@functools.lru_cache(maxsize=None)
def _make_scatter(epad, kfull, k, coff, nh, cb, t):
    zr = (nh + 16) // 16
    wr = nh // 16
    assert t >= 2 and t % 2 == 0

    def body(msg_hbm, dst_hbm, zrow_hbm, out_hbm, idx2d, r0, r1, z_v,
             s0, s1, acc):
        cid = lax.axis_index("c")
        sid = lax.axis_index("s")

        pltpu.sync_copy(zrow_hbm, z_v)
        zfull = zr // 16

        def zstep(q, carry):
            pltpu.sync_copy(z_v, acc.at[pl.ds(sid * zr + q * 16, 16)])
            return carry

        lax.fori_loop(0, zfull, zstep, 0)
        zrem = zr - zfull * 16
        if zrem:
            pltpu.sync_copy(z_v.at[pl.ds(0, zrem)],
                            acc.at[pl.ds(sid * zr + zfull * 16, zrem)])
        pltpu.sync_copy(dst_hbm.at[cid, sid], idx2d)
        plsc.subcore_barrier()

        def mread(it, buf, sem):
            base = (sid * t + it) * cb
            return pltpu.async_copy(
                msg_hbm.at[pl.ds(base, cb), pl.ds(coff, k)], buf, sem)

        def mwait(it, buf, sem):
            base = (sid * t + it) * cb
            pltpu.make_async_copy(
                msg_hbm.at[pl.ds(base, cb), pl.ds(coff, k)], buf, sem).wait()

        def sadd(it, buf):
            pltpu.sync_copy(buf, acc.at[idx2d.at[it]], add=True)

        mread(0, r0, s0)
        mread(1, r1, s1)

        def step(kk, carry):
            t1 = 2 * kk
            mwait(t1, r0, s0)
            sadd(t1, r0)
            mread(t1 + 2, r0, s0)
            mwait(t1 + 1, r1, s1)
            sadd(t1 + 1, r1)
            mread(t1 + 3, r1, s1)
            return carry

        lax.fori_loop(0, (t - 2) // 2, step, 0)
        mwait(t - 2, r0, s0)
        sadd(t - 2, r0)
        mwait(t - 1, r1, s1)
        sadd(t - 1, r1)
        plsc.subcore_barrier()

        pltpu.sync_copy(acc.at[pl.ds(sid * wr, wr)],
                        out_hbm.at[pl.ds(cid * nh + sid * wr, wr)])

    return pl.kernel(
        body,
        out_type=jax.ShapeDtypeStruct((2 * nh, k), jnp.float32),
        mesh=_mesh(),
        compiler_params=pltpu.CompilerParams(use_tc_tiling_on_sc=False),
        scratch_types=[
            pltpu.VMEM((t, cb), jnp.int32),
            pltpu.VMEM((cb, k), jnp.float32),
            pltpu.VMEM((cb, k), jnp.float32),
            pltpu.VMEM((16, k), jnp.float32),
            pltpu.SemaphoreType.DMA,
            pltpu.SemaphoreType.DMA,
            pltpu.VMEM_SHARED((nh + 16, k), jnp.float32),
        ],
    )


def _sc_scatter(msg, dst4, npad, cb, t):
    epad, k = msg.shape
    nh = npad // 2
    assert epad == NS * cb * t
    def footprint(kk):
        return ((nh + 16) * kk * 4 +
                NS * (2 * cb * kk * 4 + t * cb * 4 + 16 * kk * 4))

    nsplit = 1
    while k // nsplit > 16 and footprint(k // nsplit) > 3_600_000:
        nsplit *= 2
    kk = k // nsplit
    zrow = jnp.zeros((16, kk), jnp.float32)
    parts = [_make_scatter(epad, k, kk, si * kk, nh, cb, t)(msg, dst4, zrow)
             for si in range(nsplit)]
    if nsplit == 1:
        return parts[0]
    return jnp.concatenate(parts, axis=1)


@functools.lru_cache(maxsize=None)
def _make_msg(m_in, cpad, epad, eb):
    def body(xj_ref, q_ref, o_ref):
        q = q_ref[...]
        xs = [[xj_ref[:, (m * 2 + i) * cpad:(m * 2 + i + 1) * cpad]
               for i in range(2)] for m in range(m_in)]
        for r in range(2):
            re = None
            im = None
            for m in range(m_in):
                qr = q[:, (r * 2 + m) * 2 + 0][:, None]
                qi = q[:, (r * 2 + m) * 2 + 1][:, None]
                tr = qr * xs[m][0] - qi * xs[m][1]
                ti = qr * xs[m][1] + qi * xs[m][0]
                re = tr if re is None else re + tr
                im = ti if im is None else im + ti
            o_ref[:, (r * 2 + 0) * cpad:(r * 2 + 1) * cpad] = re
            o_ref[:, (r * 2 + 1) * cpad:(r * 2 + 2) * cpad] = im

    kx = m_in * 2 * cpad
    return pl.pallas_call(
        body,
        grid=(epad // eb,),
        in_specs=[
            pl.BlockSpec((eb, kx), lambda e: (e, 0)),
            pl.BlockSpec((eb, 8), lambda e: (e, 0)),
        ],
        out_specs=pl.BlockSpec((eb, 4 * cpad), lambda e: (e, 0)),
        out_shape=jax.ShapeDtypeStruct((epad, 4 * cpad), jnp.float32),
    )


def _tc_msg(xj, q8, m_in, cpad):
    epad = xj.shape[0]
    return _make_msg(m_in, cpad, epad, 512)(xj, q8)


@functools.lru_cache(maxsize=None)
def _make_conv_out(cpad, dpad, cres, has_res, npad, nb_rows, m_res=2):
    def body(*refs):
        if has_res:
            a_ref, w_ref, b_ref, nbias_ref, res_ref, wres_ref, o_ref = refs
        else:
            a_ref, w_ref, b_ref, nbias_ref, o_ref = refs
        w = w_ref[...]
        y = [None, None]
        for i in range(2):
            acc = jnp.broadcast_to(b_ref[...], (nb_rows, 2 * dpad)) if i == 0 \
                else jnp.zeros((nb_rows, 2 * dpad), jnp.float32)
            for r in range(2):
                a = a_ref[:, (r * 2 + i) * cpad:(r * 2 + i + 1) * cpad]
                acc = acc + jnp.dot(a, w[r * cpad:(r + 1) * cpad, :],
                                    preferred_element_type=jnp.float32)
            y[i] = acc
        if has_res:
            wres = wres_ref[...]
            for i in range(2):
                parts = []
                for o in range(2):
                    oe = min(o, m_res - 1)
                    rm = res_ref[:, (oe * 2 + i) * cres:(oe * 2 + i + 1) * cres]
                    parts.append(jnp.dot(rm, wres,
                                         preferred_element_type=jnp.float32))
                y[i] = y[i] + jnp.concatenate(parts, axis=1)
        sq = y[0] * y[0] + y[1] * y[1]
        mag = jnp.sqrt(sq + 1e-12)
        scale = jax.nn.relu(mag + nbias_ref[...]) / (mag + 1e-6)
        for o in range(2):
            for i in range(2):
                o_ref[:, (o * 2 + i) * dpad:(o * 2 + i + 1) * dpad] = \
                    (y[i] * scale)[:, o * dpad:(o + 1) * dpad]

    in_specs = [
        pl.BlockSpec((nb_rows, 4 * cpad), lambda n: (n, 0)),
        pl.BlockSpec((2 * cpad, 2 * dpad), lambda n: (0, 0)),
        pl.BlockSpec((1, 2 * dpad), lambda n: (0, 0)),
        pl.BlockSpec((1, 2 * dpad), lambda n: (0, 0)),
    ]
    if has_res:
        in_specs += [
            pl.BlockSpec((nb_rows, 2 * m_res * cres), lambda n: (n, 0)),
            pl.BlockSpec((cres, dpad), lambda n: (0, 0)),
        ]
    return pl.pallas_call(
        body,
        grid=(npad // nb_rows,),
        in_specs=in_specs,
        out_specs=pl.BlockSpec((nb_rows, 4 * dpad), lambda n: (n, 0)),
        out_shape=jax.ShapeDtypeStruct((npad, 4 * dpad), jnp.float32),
    )


def _tc_conv_out(agg, w, b2, nb2, res=None, wres=None):
    npad, ka = agg.shape
    cpad = ka // 4
    dpad = w.shape[1] // 2
    if res is not None:
        cres = wres.shape[0]
        m_res = res.shape[1] // (2 * cres)
        return _make_conv_out(cpad, dpad, cres, True, npad, 256, m_res)(
            agg, w, b2, nb2, res, wres)
    return _make_conv_out(cpad, dpad, 0, False, npad, 256)(agg, w, b2, nb2)


def _head_body(x_ref, w_ref, b_ref, o_ref):
    w = w_ref[...]
    b = b_ref[...]
    acc = None
    for m in range(2):
        xr = x_ref[:, (m * 2 + 0) * 32:(m * 2 + 1) * 32]
        xi = x_ref[:, (m * 2 + 1) * 32:(m * 2 + 2) * 32]
        yr = jnp.dot(xr, w, preferred_element_type=jnp.float32)
        yi = jnp.dot(xi, w, preferred_element_type=jnp.float32)
        sq = yr * yr + yi * yi
        mag = jnp.sqrt(sq + 1e-12)
        scale = jax.nn.relu(mag + b) / (mag + 1e-6)
        mag2 = jnp.sqrt(scale * scale * sq + 1e-12)
        acc = mag2 if acc is None else acc + mag2
    o_ref[...] = acc


def _head(x_planar, w_pad, b_pad):
    n, _ = x_planar.shape
    jp = w_pad.shape[1]
    nb = 512
    return pl.pallas_call(
        _head_body,
        grid=(n // nb,),
        in_specs=[
            pl.BlockSpec((nb, 128), lambda i: (i, 0)),
            pl.BlockSpec((32, jp), lambda i: (0, 0)),
            pl.BlockSpec((1, jp), lambda i: (0, 0)),
        ],
        out_specs=pl.BlockSpec((nb, jp), lambda i: (i, 0)),
        out_shape=jax.ShapeDtypeStruct((n, jp), jnp.float32),
    )(x_planar, w_pad, b_pad)


def _pad_rows(a, rows, fill=0):
    pad = rows - a.shape[0]
    if pad == 0:
        return a
    return jnp.concatenate(
        [a, jnp.full((pad,) + a.shape[1:], fill, a.dtype)], axis=0)


def _prep_w(w, cin, cout, cpad, dpad):
    wp = jnp.zeros((2, 2, cpad, dpad), jnp.float32)
    wp = wp.at[:, :, :cin, :cout].set(w)
    return wp.transpose(1, 2, 0, 3).reshape(2 * cpad, 2 * dpad)


def _prep_b(b, cout, dpad):
    bp = jnp.zeros((dpad,), jnp.float32).at[:cout].set(b)
    return jnp.concatenate([bp, bp])[None]


def _norm2(v):
    return v / (jnp.linalg.norm(v, axis=-1, keepdims=True) + 1e-8)


def _rotate_rows(tab, pc, cpad):
    n = tab.shape[0]
    x = tab.reshape(n, 2, 2, cpad)
    pr = pc[:, 0][:, None, None]
    pi = pc[:, 1][:, None, None]
    re = x[:, :, 0] * pr - x[:, :, 1] * pi
    im = x[:, :, 1] * pr + x[:, :, 0] * pi
    return jnp.stack([re, im], axis=2).reshape(n, 4 * cpad)


_LEVEL = {
    0: dict(n=10000, npad=10240, e=160000, epad=163840, scb=64, st=160),
    1: dict(n=5000, npad=5120, e=80000, epad=81920, scb=64, st=80),
    2: dict(n=2500, npad=2560, e=40000, epad=40960, scb=64, st=40),
    3: dict(n=1250, npad=1280, e=20000, epad=20480, scb=64, st=20),
}
_IDXG = {5120: (5120, 16, 10), 2560: (2560, 8, 10), 1280: (2048, 8, 8),
         10240: (10240, 32, 10)}


def _idx_gather(tab, idx_raw, rows_target):
    gpad, cb, t = _IDXG[rows_target]
    idx = _pad_rows(idx_raw.astype(jnp.int32), gpad, 0)
    g = _sc_gather(tab, idx, cb, t)
    return g[:rows_target]


def _forward(p):
    pos = p['pos']

    levels = {}
    for s in range(4):
        lv = dict(_LEVEL[s])
        ei = p['edge_index%d' % s]
        src = _pad_rows(ei[0].astype(jnp.int32), lv['epad'], 0)
        dst = _pad_rows(ei[1].astype(jnp.int32), lv['epad'], -1)
        nh = lv['npad'] // 2
        loc = []
        for c in (0, 1):
            l = dst - c * nh
            ok = (l >= 0) & (l < nh)
            loc.append(jnp.where(ok, l, nh))
        lv['dst4'] = jnp.stack(loc).reshape(2, NS, lv['st'], lv['scb'])
        cn = _norm2(p['connection%d' % s])
        pre = p['precomp%d' % s]
        q = jnp.zeros((lv['e'], 2, 2, 2), jnp.float32)
        q = q.at[:, :, 0, :].set(pre)
        qr = pre[:, :, 0] * cn[:, None, 0] - pre[:, :, 1] * cn[:, None, 1]
        qi = pre[:, :, 0] * cn[:, None, 1] + pre[:, :, 1] * cn[:, None, 0]
        q = q.at[:, :, 1, 0].set(qr).at[:, :, 1, 1].set(qi)
        lv['src'] = src
        lv['dst'] = dst
        lv['q8'] = _pad_rows(q.reshape(lv['e'], 8), lv['epad'])
        levels[s] = lv

    def conv(x_tab, s, m_in, cin, cout, wkey, bkey, nbkey):
        lv = levels[s]
        cpad = max(8, cin)
        dpad = cout
        xj = _sc_gather(x_tab, lv['src'])
        msg = _tc_msg(xj, lv['q8'], m_in, cpad)
        agg = _sc_scatter(msg, lv['dst4'], lv['npad'], lv['scb'], lv['st'])
        w = _prep_w(p[wkey], cin, cout, cpad, dpad)
        return agg, w, _prep_b(p[bkey], cout, dpad), _prep_b(p[nbkey], cout, dpad)

    def res_block(x_tab, s, name, cin, cout, m_in=2):
        cpad_in = max(8, cin)
        agg, w1, b1, nb1 = conv(x_tab, s, m_in, cin, cout,
                                name + '_W1', name + '_b1', name + '_nb1')
        h = _tc_conv_out(agg, w1, b1, nb1)
        agg2, w2, b2, nb2 = conv(h, s, 2, cout, cout,
                                 name + '_W2', name + '_b2', name + '_nb2')
        if name + '_Wres' in p:
            wres = jnp.zeros((cpad_in, cout), jnp.float32)
            wres = wres.at[:cin, :].set(p[name + '_Wres'])
        else:
            wres = jnp.eye(cpad_in, cout, dtype=jnp.float32)
        return _tc_conv_out(agg2, w2, b2, nb2, res=x_tab, wres=wres)

    def pool(x_tab, l, ncoarse_pad):
        g = _idx_gather(x_tab, p['pool_idx%d' % l], ncoarse_pad)
        pc = _pad_rows(_norm2(p['pool_conn%d' % l]), ncoarse_pad, 0)
        return _rotate_rows(g, pc, x_tab.shape[1] // 4)

    def unpool(x_coarse, xp_tab, idx_raw, nfine_pad):
        up = _idx_gather(x_coarse, idx_raw, nfine_pad)
        n = nfine_pad
        ca = up.shape[1] // 4
        cb2 = xp_tab.shape[1] // 4
        cat = jnp.concatenate([up.reshape(n, 4, ca), xp_tab.reshape(n, 4, cb2)],
                              axis=2)
        return cat.reshape(n, 4 * (ca + cb2))

    x0 = jnp.zeros((_LEVEL[0]['npad'], 16), jnp.float32)
    x0 = x0.at[:10000, 0:3].set(pos)

    x = res_block(x0, 0, 'b01', 3, 16, m_in=1)
    x = res_block(x, 0, 'b11', 16, 32)
    xp1 = res_block(x, 0, 'b12', 32, 32)
    x = pool(xp1, 1, 5120)
    x = res_block(x, 1, 'b21', 32, 64)
    xp2 = res_block(x, 1, 'b22', 64, 64)
    x = pool(xp2, 2, 2560)
    x = res_block(x, 2, 'b31', 64, 64)
    xp3 = res_block(x, 2, 'b32', 64, 64)
    x = pool(xp3, 3, 1280)
    x = res_block(x, 3, 'b41', 64, 64)
    x = res_block(x, 3, 'b42', 64, 64)
    x = res_block(x, 3, 'b51', 64, 64)
    x = res_block(x, 3, 'b52', 64, 64)
    x = unpool(x, xp3, p['unpool3'], 2560)
    x = res_block(x, 2, 'b61', 128, 64)
    x = res_block(x, 2, 'b62', 64, 64)
    x = unpool(x, xp2, p['unpool2'], 5120)
    x = res_block(x, 1, 'b71', 128, 32)
    x = res_block(x, 1, 'b72', 32, 32)
    x = unpool(x, xp1, p['unpool1'], 10240)
    x = res_block(x, 0, 'b81', 64, 32)
    x = res_block(x, 0, 'b82', 32, 32)

    jp = 384
    w_pad = jnp.zeros((32, jp), jnp.float32).at[:, :300].set(p['lin1_W'])
    b_pad = jnp.zeros((1, jp), jnp.float32).at[0, :300].set(p['nonlin1_b'])
    out = _head(x, w_pad, b_pad)[:10000, :300]
    return out[None], pos[None]


def kernel(pos, edge_index0, precomp0, connection0, edge_index1, precomp1, connection1, edge_index2, precomp2, connection2, edge_index3, precomp3, connection3, pool_idx1, pool_conn1, pool_idx2, pool_conn2, pool_idx3, pool_conn3, unpool3, unpool2, unpool1, b01_W1, b01_b1, b01_nb1, b01_W2, b01_b2, b01_nb2, b01_Wres, b11_W1, b11_b1, b11_nb1, b11_W2, b11_b2, b11_nb2, b11_Wres, b12_W1, b12_b1, b12_nb1, b12_W2, b12_b2, b12_nb2, b21_W1, b21_b1, b21_nb1, b21_W2, b21_b2, b21_nb2, b21_Wres, b22_W1, b22_b1, b22_nb1, b22_W2, b22_b2, b22_nb2, b31_W1, b31_b1, b31_nb1, b31_W2, b31_b2, b31_nb2, b32_W1, b32_b1, b32_nb1, b32_W2, b32_b2, b32_nb2, b41_W1, b41_b1, b41_nb1, b41_W2, b41_b2, b41_nb2, b42_W1, b42_b1, b42_nb1, b42_W2, b42_b2, b42_nb2, b51_W1, b51_b1, b51_nb1, b51_W2, b51_b2, b51_nb2, b52_W1, b52_b1, b52_nb1, b52_W2, b52_b2, b52_nb2, b61_W1, b61_b1, b61_nb1, b61_W2, b61_b2, b61_nb2, b61_Wres, b62_W1, b62_b1, b62_nb1, b62_W2, b62_b2, b62_nb2, b71_W1, b71_b1, b71_nb1, b71_W2, b71_b2, b71_nb2, b71_Wres, b72_W1, b72_b1, b72_nb1, b72_W2, b72_b2, b72_nb2, b81_W1, b81_b1, b81_nb1, b81_W2, b81_b2, b81_nb2, b81_Wres, b82_W1, b82_b1, b82_nb1, b82_W2, b82_b2, b82_nb2, lin1_W, nonlin1_b):
    return _forward(dict(locals()))
